# baseline (speedup 1.0000x reference)
import functools

import jax
import jax.numpy as jnp
from jax import lax
from jax.experimental import pallas as pl
from jax.experimental.pallas import tpu as pltpu
from jax.experimental.pallas import tpu_sc as plsc

B = 16384
L = 50
D = 64
CLS = 10
NC = 2
NS = 16
NW = NC * NS
ROWS_PER_W = B // NW
G = 2
IDX_PER_G = G * L
NGROUPS = ROWS_PER_W // G
NBUF = 4
DREG = D // 16


def _sc_gather_sum(table, texts_r):
    mesh = plsc.VectorSubcoreMesh(core_axis_name="c", subcore_axis_name="s")

    @functools.partial(
        pl.kernel,
        out_type=jax.ShapeDtypeStruct((B, D), jnp.float32),
        mesh=mesh,
        scratch_types=(
            [pltpu.VMEM((NGROUPS, IDX_PER_G), jnp.int32),
             pltpu.VMEM((NBUF, IDX_PER_G, D), jnp.float32),
             pltpu.VMEM((ROWS_PER_W, D), jnp.float32)]
            + [pltpu.SemaphoreType.DMA] * NBUF
        ),
        compiler_params=pltpu.CompilerParams(use_tc_tiling_on_sc=False),
    )
    def k(table_hbm, texts_hbm, out_hbm, idx_v, rows_v, out_v, *sems):
        w = lax.axis_index("s") * NC + lax.axis_index("c")
        pltpu.sync_copy(texts_hbm.at[w], idx_v)
        for b in range(NBUF):
            pltpu.async_copy(table_hbm.at[idx_v.at[b]], rows_v.at[b], sems[b])

        def body(g0, carry):
            for b in range(NBUF):
                g = g0 * NBUF + b
                pltpu.make_async_copy(
                    table_hbm.at[idx_v.at[g]], rows_v.at[b], sems[b]).wait()
                for i in range(G):
                    accs = [rows_v[b, i * L, pl.ds(d * 16, 16)]
                            for d in range(DREG)]
                    for r in range(1, L):
                        for d in range(DREG):
                            accs[d] = accs[d] + rows_v[b, i * L + r,
                                                       pl.ds(d * 16, 16)]
                    row = G * g + i
                    for d in range(DREG):
                        out_v[row, pl.ds(d * 16, 16)] = accs[d]
                gn = g + NBUF

                @pl.when(gn < NGROUPS)
                def _():
                    pltpu.async_copy(
                        table_hbm.at[idx_v.at[gn]], rows_v.at[b], sems[b])
            return carry

        lax.fori_loop(0, NGROUPS // NBUF, body, 0)
        pltpu.sync_copy(out_v, out_hbm.at[pl.ds(w * ROWS_PER_W, ROWS_PER_W)])

    return k(table, texts_r)


def _tc_fc(x, wt, bias2d):
    tb = 2048

    def body(x_ref, w_ref, b_ref, o_ref):
        o_ref[...] = (
            jnp.dot(x_ref[...], w_ref[...], preferred_element_type=jnp.float32)
            * (1.0 / L)
            + b_ref[...]
        )

    return pl.pallas_call(
        body,
        grid=(B // tb,),
        in_specs=[
            pl.BlockSpec((tb, D), lambda i: (i, 0)),
            pl.BlockSpec((D, CLS), lambda i: (0, 0)),
            pl.BlockSpec((1, CLS), lambda i: (0, 0)),
        ],
        out_specs=pl.BlockSpec((tb, CLS), lambda i: (i, 0)),
        out_shape=jax.ShapeDtypeStruct((B, CLS), jnp.float32),
    )(x, wt, bias2d)


V = 1000000
TBLK = 1024
NVB = (V + TBLK - 1) // TBLK
V2 = NVB * TBLK


def _tc_relayout(table_t):
    def body(x_ref, o_ref):
        t = x_ref[...]
        o_ref[...] = jnp.concatenate([t[:TBLK // 2], t[TBLK // 2:]], axis=1)

    return pl.pallas_call(
        body,
        grid=(NVB,),
        in_specs=[pl.BlockSpec((TBLK, D), lambda i: (i, 0))],
        out_specs=pl.BlockSpec((TBLK // 2, 2 * D), lambda i: (i, 0)),
        out_shape=jax.ShapeDtypeStruct((V2 // 2, 2 * D), jnp.float32),
    )(table_t)


def kernel(texts, emb_table, fc_weight, fc_bias):
    v = texts.astype(jnp.int32)
    r = (v & ~(TBLK - 1)) + 2 * (v & (TBLK // 2 - 1)) + (
        (v >> 9) & 1)
    idx = r.reshape(NW, NGROUPS, IDX_PER_G)
    tblp = _tc_relayout(emb_table)
    sums = _sc_gather_sum(tblp.reshape(V2, D), idx)
    return _tc_fc(sums, fc_weight.T, fc_bias.reshape(1, CLS))


# speedup vs baseline: 1.5238x; 1.5238x over previous
import functools

import jax
import jax.numpy as jnp
from jax import lax
from jax.experimental import pallas as pl
from jax.experimental.pallas import tpu as pltpu
from jax.experimental.pallas import tpu_sc as plsc

B = 16384
L = 50
D = 64
CLS = 10
NC = 2
NS = 16
NW = NC * NS
ROWS_PER_W = B // NW
G = 2
IDX_PER_G = G * L
NGROUPS = ROWS_PER_W // G
NBUF = 4
DREG = D // 16


def _sc_gather_sum(table, texts_r):
    mesh = plsc.VectorSubcoreMesh(core_axis_name="c", subcore_axis_name="s")

    @functools.partial(
        pl.kernel,
        out_type=jax.ShapeDtypeStruct((B, D), jnp.float32),
        mesh=mesh,
        scratch_types=(
            [pltpu.VMEM((NGROUPS, IDX_PER_G), jnp.int32),
             pltpu.VMEM((NBUF, IDX_PER_G, D), jnp.float32),
             pltpu.VMEM((ROWS_PER_W, D), jnp.float32)]
            + [pltpu.SemaphoreType.DMA] * NBUF
        ),
        compiler_params=pltpu.CompilerParams(use_tc_tiling_on_sc=False),
    )
    def k(table_hbm, texts_hbm, out_hbm, idx_v, rows_v, out_v, *sems):
        w = lax.axis_index("s") * NC + lax.axis_index("c")
        pltpu.sync_copy(texts_hbm.at[w], idx_v)
        for b in range(NBUF):
            pltpu.async_copy(table_hbm.at[idx_v.at[b]], rows_v.at[b], sems[b])

        def body(g0, carry):
            for b in range(NBUF):
                g = g0 * NBUF + b
                pltpu.make_async_copy(
                    table_hbm.at[idx_v.at[g]], rows_v.at[b], sems[b]).wait()
                for i in range(G):
                    accs = [rows_v[b, i * L, pl.ds(d * 16, 16)]
                            for d in range(DREG)]
                    for r in range(1, L):
                        for d in range(DREG):
                            accs[d] = accs[d] + rows_v[b, i * L + r,
                                                       pl.ds(d * 16, 16)]
                    row = G * g + i
                    for d in range(DREG):
                        out_v[row, pl.ds(d * 16, 16)] = accs[d]
                gn = g + NBUF

                @pl.when(gn < NGROUPS)
                def _():
                    pltpu.async_copy(
                        table_hbm.at[idx_v.at[gn]], rows_v.at[b], sems[b])
            return carry

        lax.fori_loop(0, NGROUPS // NBUF, body, 0)
        pltpu.sync_copy(out_v, out_hbm.at[pl.ds(w * ROWS_PER_W, ROWS_PER_W)])

    return k(table, texts_r)


def _tc_fc(x, wt, bias2d):
    tb = 2048

    def body(x_ref, w_ref, b_ref, o_ref):
        o_ref[...] = (
            jnp.dot(x_ref[...], w_ref[...], preferred_element_type=jnp.float32)
            * (1.0 / L)
            + b_ref[...]
        )

    return pl.pallas_call(
        body,
        grid=(B // tb,),
        in_specs=[
            pl.BlockSpec((tb, D), lambda i: (i, 0)),
            pl.BlockSpec((D, CLS), lambda i: (0, 0)),
            pl.BlockSpec((1, CLS), lambda i: (0, 0)),
        ],
        out_specs=pl.BlockSpec((tb, CLS), lambda i: (i, 0)),
        out_shape=jax.ShapeDtypeStruct((B, CLS), jnp.float32),
    )(x, wt, bias2d)


def kernel(texts, emb_table, fc_weight, fc_bias):
    idx = texts.astype(jnp.int32).reshape(NW, NGROUPS, IDX_PER_G)
    sums = _sc_gather_sum(emb_table, idx)
    return _tc_fc(sums, fc_weight.T, fc_bias.reshape(1, CLS))
